# baseline (speedup 1.0000x reference)
import math

import jax
import jax.numpy as jnp
from jax.experimental import pallas as pl
from jax.experimental.pallas import tpu as pltpu

N = 10000
E = 320000
D = 128
H = 8
Dh = D // H
CUTOFF = 5.0

_NBLK = 400


def _qkv_body(x_ref, wq_ref, bq_ref, wk_ref, bk_ref, wv_ref, bv_ref,
              q_ref, k_ref, v_ref):
    x = x_ref[...]
    q_ref[...] = jnp.dot(x, wq_ref[...], preferred_element_type=jnp.float32) + bq_ref[...]
    k_ref[...] = jnp.dot(x, wk_ref[...], preferred_element_type=jnp.float32) + bk_ref[...]
    v_ref[...] = jnp.dot(x, wv_ref[...], preferred_element_type=jnp.float32) + bv_ref[...]


def _qkv(x, Wq, bq, Wk, bk, Wv, bv):
    grid = (N // _NBLK,)
    blk = pl.BlockSpec((_NBLK, D), lambda i: (i, 0))
    wblk = pl.BlockSpec((D, D), lambda i: (0, 0))
    bblk = pl.BlockSpec((1, D), lambda i: (0, 0))
    out = jax.ShapeDtypeStruct((N, D), jnp.float32)
    return pl.pallas_call(
        _qkv_body,
        grid=grid,
        in_specs=[blk, wblk, bblk, wblk, bblk, wblk, bblk],
        out_specs=[blk, blk, blk],
        out_shape=[out, out, out],
    )(x, Wq, bq.reshape(1, D), Wk, bk.reshape(1, D), Wv, bv.reshape(1, D))


def _final_body(acc_ref, x_ref, wo_ref, bo_ref, g_ref, b_ref, y_ref):
    o = jnp.dot(acc_ref[...], wo_ref[...], preferred_element_type=jnp.float32)
    y = o + bo_ref[...] + x_ref[...]
    mu = jnp.mean(y, axis=-1, keepdims=True)
    yc = y - mu
    var = jnp.mean(yc * yc, axis=-1, keepdims=True)
    yn = yc * jax.lax.rsqrt(var + 1e-05)
    y_ref[...] = yn * g_ref[...] + b_ref[...]


def _final(acc, x, Wo, bo, gamma, beta):
    grid = (N // _NBLK,)
    blk = pl.BlockSpec((_NBLK, D), lambda i: (i, 0))
    wblk = pl.BlockSpec((D, D), lambda i: (0, 0))
    bblk = pl.BlockSpec((1, D), lambda i: (0, 0))
    return pl.pallas_call(
        _final_body,
        grid=grid,
        in_specs=[blk, blk, wblk, bblk, bblk, bblk],
        out_specs=blk,
        out_shape=jax.ShapeDtypeStruct((N, D), jnp.float32),
    )(acc, x, Wo, bo.reshape(1, D), gamma.reshape(1, D), beta.reshape(1, D))


def kernel(x, edge_index, edge_vec, edge_length, Wq, bq, Wk, bk, Wv, bv,
           W1, b1, W2, b2, Wo, bo, gamma, beta):
    row = edge_index[0]
    col = edge_index[1]
    q, k, v = _qkv(x, Wq, bq, Wk, bk, Wv, bv)
    q = q.reshape(N, H, Dh)
    k = k.reshape(N, H, Dh)
    v = v.reshape(N, H, Dh)

    q_i = q[row]
    k_j = k[col]
    attn = jnp.sum(q_i * k_j, axis=-1) / math.sqrt(Dh)
    h = jax.nn.silu(edge_length @ W1 + b1)
    edge_bias = h @ W2 + b2
    attn = attn + edge_bias
    cut = 0.5 * (jnp.cos(edge_length * math.pi / CUTOFF) + 1.0)
    cut = cut * (edge_length < CUTOFF).astype(jnp.float32)
    attn = attn * cut
    attn_exp = jnp.exp(attn - jnp.max(attn, axis=0, keepdims=True))
    attn_sum = jax.ops.segment_sum(attn_exp, row, num_segments=N)
    attn_n = attn_exp / (attn_sum[row] + 1e-08)
    out = jax.ops.segment_sum(attn_n[:, :, None] * v[col], row, num_segments=N)

    return _final(out.reshape(N, D), x, Wo, bo, gamma, beta)


# speedup vs baseline: 8.3135x; 8.3135x over previous
import functools
import math

import jax
import jax.numpy as jnp
from jax import lax
from jax.experimental import pallas as pl
from jax.experimental.pallas import tpu as pltpu
from jax.experimental.pallas import tpu_sc as plsc

N = 10000
E = 320000
D = 128
H = 8
Dh = D // H
CUTOFF = 5.0

NC = 2
NS = 16
NW = NC * NS
L = 16

EPW = E // NW
C = 80
NCH = EPW // C
NJ = E // C
SUMW = N * H
CH = C * H

_RED_W = 20
_RED_SL = SUMW // _RED_W

HH = H // NC
DC = D // NC
EPT = E // NS
NCH5 = EPT // C
_ZR = 80

_mesh = plsc.VectorSubcoreMesh(
    core_axis_name="c", subcore_axis_name="s", num_cores=NC, num_subcores=NS)
_sc_params = pltpu.CompilerParams(
    needs_layout_passes=False, use_tc_tiling_on_sc=False)


def _wid():
    return lax.axis_index("s") * NC + lax.axis_index("c")


_NBLK = 400


def _qkv_body(x_ref, wq_ref, bq_ref, wk_ref, bk_ref, wv_ref, bv_ref,
              q_ref, k_ref, v_ref):
    x = x_ref[...]
    q_ref[...] = jnp.dot(x, wq_ref[...], preferred_element_type=jnp.float32) + bq_ref[...]
    k_ref[...] = jnp.dot(x, wk_ref[...], preferred_element_type=jnp.float32) + bk_ref[...]
    v_ref[...] = jnp.dot(x, wv_ref[...], preferred_element_type=jnp.float32) + bv_ref[...]


def _qkv(x, Wq, bq, Wk, bk, Wv, bv):
    blk = pl.BlockSpec((_NBLK, D), lambda i: (i, 0))
    wblk = pl.BlockSpec((D, D), lambda i: (0, 0))
    bblk = pl.BlockSpec((1, D), lambda i: (0, 0))
    out = jax.ShapeDtypeStruct((N, D), jnp.float32)
    return pl.pallas_call(
        _qkv_body,
        grid=(N // _NBLK,),
        in_specs=[blk, wblk, bblk, wblk, bblk, wblk, bblk],
        out_specs=[blk, blk, blk],
        out_shape=[out, out, out],
    )(x, Wq, bq.reshape(1, D), Wk, bk.reshape(1, D), Wv, bv.reshape(1, D))


_EB = 2000


def _edge_body(len_ref, w1_ref, b1_ref, w2_ref, b2_ref, c0_ref, c1_ref):
    ln = len_ref[...]
    hid = jax.nn.silu(ln * w1_ref[...] + b1_ref[...])
    bias = jnp.dot(hid, w2_ref[...], preferred_element_type=jnp.float32) + b2_ref[...]
    cut = 0.5 * (jnp.cos(ln * (math.pi / CUTOFF)) + 1.0)
    cut = cut * (ln < CUTOFF).astype(jnp.float32)
    c0_ref[...] = bias * cut
    c1_ref[...] = cut * (1.0 / math.sqrt(Dh))


def _edge_coeffs(edge_length, W1, b1, W2, b2):
    lblk = pl.BlockSpec((_EB, 1), lambda i: (i, 0))
    w1blk = pl.BlockSpec((1, D), lambda i: (0, 0))
    w2blk = pl.BlockSpec((D, H), lambda i: (0, 0))
    b2blk = pl.BlockSpec((1, H), lambda i: (0, 0))
    return pl.pallas_call(
        _edge_body,
        grid=(E // _EB,),
        in_specs=[lblk, w1blk, w1blk, w2blk, b2blk],
        out_specs=[pl.BlockSpec((_EB, H), lambda i: (i, 0)), lblk],
        out_shape=[jax.ShapeDtypeStruct((E, H), jnp.float32),
                   jax.ShapeDtypeStruct((E, 1), jnp.float32)],
    )(edge_length, W1, b1.reshape(1, D), W2, b2.reshape(1, H))


def _final_body(acc_ref, x_ref, wo_ref, bo_ref, g_ref, b_ref, y_ref):
    o = jnp.dot(acc_ref[...], wo_ref[...], preferred_element_type=jnp.float32)
    y = o + bo_ref[...] + x_ref[...]
    mu = jnp.mean(y, axis=-1, keepdims=True)
    yc = y - mu
    var = jnp.mean(yc * yc, axis=-1, keepdims=True)
    yn = yc * lax.rsqrt(var + 1e-05)
    y_ref[...] = yn * g_ref[...] + b_ref[...]


def _final(acc, x, Wo, bo, gamma, beta):
    blk = pl.BlockSpec((_NBLK, D), lambda i: (i, 0))
    wblk = pl.BlockSpec((D, D), lambda i: (0, 0))
    bblk = pl.BlockSpec((1, D), lambda i: (0, 0))
    return pl.pallas_call(
        _final_body,
        grid=(N // _NBLK,),
        in_specs=[blk, blk, wblk, bblk, bblk, bblk],
        out_specs=blk,
        out_shape=jax.ShapeDtypeStruct((N, D), jnp.float32),
    )(acc, x, Wo, bo.reshape(1, D), gamma.reshape(1, D), beta.reshape(1, D))


@functools.partial(
    pl.kernel,
    out_type=[jax.ShapeDtypeStruct((E * H,), jnp.float32),
              jax.ShapeDtypeStruct((NW * H * L,), jnp.float32)],
    mesh=_mesh,
    compiler_params=_sc_params,
    scratch_types=[
        pltpu.VMEM((C,), jnp.int32),
        pltpu.VMEM((C,), jnp.int32),
        pltpu.VMEM((C, D), jnp.float32),
        pltpu.VMEM((C, D), jnp.float32),
        pltpu.VMEM((CH,), jnp.float32),
        pltpu.VMEM((CH,), jnp.float32),
        pltpu.VMEM((C,), jnp.float32),
        pltpu.VMEM((H * L,), jnp.float32),
        pltpu.SemaphoreType.DMA,
        pltpu.SemaphoreType.DMA,
    ],
)
def _k1(q_hbm, k_hbm, row_hbm, col_hbm, c0_hbm, c1_hbm,
        a_hbm, wmax_hbm,
        rowi, coli, qrows, krows, abuf, c0buf, c1buf, wmaxb, semq, semk):
    w = _wid()
    iota = lax.broadcasted_iota(jnp.int32, (L,), 0)
    neg = jnp.full((L,), -3.0e38, jnp.float32)
    for h in range(H):
        wmaxb[pl.ds(h * L, L)] = neg

    def chunk(ci, carry):
        base = w * EPW + ci * C
        pltpu.sync_copy(row_hbm.at[pl.ds(base, C)], rowi)
        pltpu.sync_copy(col_hbm.at[pl.ds(base, C)], coli)
        cpq = pltpu.async_copy(q_hbm.at[rowi], qrows, semq)
        cpk = pltpu.async_copy(k_hbm.at[coli], krows, semk)
        pltpu.sync_copy(c0_hbm.at[pl.ds(base * H, CH)], c0buf)
        pltpu.sync_copy(c1_hbm.at[pl.ds(base, C)], c1buf)
        cpq.wait()
        cpk.wait()
        for g in range(C // L):
            el = iota + (g * L)
            el8 = el * H
            c1v = c1buf[pl.ds(g * L, L)]
            for h in range(H):
                def dstep(d, acc):
                    dv = jnp.full((L,), h * Dh, jnp.int32) + d
                    qv = plsc.load_gather(qrows, [el, dv])
                    kv = plsc.load_gather(krows, [el, dv])
                    return acc + qv * kv
                dot = lax.fori_loop(0, Dh, dstep, jnp.zeros((L,), jnp.float32),
                                    unroll=4)
                c0v = plsc.load_gather(c0buf, [el8 + h])
                a = dot * c1v + c0v
                abuf[pl.ds(h * C + g * L, L)] = a
                wmaxb[pl.ds(h * L, L)] = jnp.maximum(wmaxb[pl.ds(h * L, L)], a)
        pltpu.sync_copy(abuf, a_hbm.at[pl.ds((w * NCH + ci) * CH, CH)])
        return carry

    lax.fori_loop(0, NCH, chunk, 0)
    pltpu.sync_copy(wmaxb, wmax_hbm.at[pl.ds(w * H * L, H * L)])


@functools.partial(
    pl.kernel,
    out_type=[jax.ShapeDtypeStruct((E * H,), jnp.float32),
              jax.ShapeDtypeStruct((NW * SUMW,), jnp.float32)],
    mesh=_mesh,
    compiler_params=_sc_params,
    scratch_types=[
        pltpu.VMEM((C,), jnp.int32),
        pltpu.VMEM((CH,), jnp.float32),
        pltpu.VMEM((CH,), jnp.float32),
        pltpu.VMEM((NW * H * L,), jnp.float32),
        pltpu.VMEM((SUMW,), jnp.float32),
    ],
)
def _k3(row_hbm, a_hbm, wmax_hbm,
        p_hbm, sums_hbm,
        rowi, abuf, pbuf, wmaxall, sums):
    w = _wid()
    pltpu.sync_copy(wmax_hbm, wmaxall)
    m = []
    for h in range(H):
        acc = wmaxall[pl.ds(h * L, L)]
        for t in range(1, NW):
            acc = jnp.maximum(acc, wmaxall[pl.ds(t * H * L + h * L, L)])
        m.append(jnp.max(acc))

    zero16 = jnp.zeros((L,), jnp.float32)

    def zstep(i, carry):
        sums[pl.ds(i * L, L)] = zero16
        return carry

    lax.fori_loop(0, SUMW // L, zstep, 0)

    def chunk(ci, carry):
        base = w * EPW + ci * C
        pltpu.sync_copy(row_hbm.at[pl.ds(base, C)], rowi)
        pltpu.sync_copy(a_hbm.at[pl.ds((w * NCH + ci) * CH, CH)], abuf)
        for g in range(C // L):
            rbase = rowi[pl.ds(g * L, L)] * H
            for h in range(H):
                p = jnp.exp(abuf[pl.ds(h * C + g * L, L)] - m[h])
                pbuf[pl.ds(h * C + g * L, L)] = p
                plsc.addupdate_scatter(sums, [rbase + h], p)
        pltpu.sync_copy(pbuf, p_hbm.at[pl.ds((w * NCH + ci) * CH, CH)])
        return carry

    lax.fori_loop(0, NCH, chunk, 0)
    pltpu.sync_copy(sums, sums_hbm.at[pl.ds(w * SUMW, SUMW)])


@functools.partial(
    pl.kernel,
    out_type=jax.ShapeDtypeStruct((SUMW,), jnp.float32),
    mesh=_mesh,
    compiler_params=_sc_params,
    scratch_types=[
        pltpu.VMEM((_RED_SL,), jnp.float32),
        pltpu.VMEM((_RED_SL,), jnp.float32),
    ],
)
def _k4a(sums_hbm, inv_hbm, acc, tbuf):
    w = _wid()

    @pl.when(w < _RED_W)
    def _():
        base = w * _RED_SL
        ng = _RED_SL // L

        def zstep(i, carry):
            acc[pl.ds(i * L, L)] = jnp.zeros((L,), jnp.float32)
            return carry

        lax.fori_loop(0, ng, zstep, 0)

        def tstep(t, carry):
            pltpu.sync_copy(sums_hbm.at[pl.ds(t * SUMW + base, _RED_SL)], tbuf)

            def astep(i, c2):
                acc[pl.ds(i * L, L)] = acc[pl.ds(i * L, L)] + tbuf[pl.ds(i * L, L)]
                return c2

            lax.fori_loop(0, ng, astep, 0)
            return carry

        lax.fori_loop(0, NW, tstep, 0)

        def istep(i, carry):
            acc[pl.ds(i * L, L)] = 1.0 / (acc[pl.ds(i * L, L)] + 1e-08)
            return carry

        lax.fori_loop(0, ng, istep, 0)
        pltpu.sync_copy(acc, inv_hbm.at[pl.ds(base, _RED_SL)])


@functools.partial(
    pl.kernel,
    out_type=jax.ShapeDtypeStruct((E * H,), jnp.float32),
    mesh=_mesh,
    compiler_params=_sc_params,
    scratch_types=[
        pltpu.VMEM((C,), jnp.int32),
        pltpu.VMEM((CH,), jnp.float32),
        pltpu.VMEM((CH,), jnp.float32),
        pltpu.VMEM((SUMW,), jnp.float32),
    ],
)
def _k4b(row_hbm, p_hbm, inv_hbm, pn_hbm, rowi, pbuf, pnbuf, invb):
    w = _wid()
    pltpu.sync_copy(inv_hbm, invb)

    def chunk(ci, carry):
        base = w * EPW + ci * C
        pltpu.sync_copy(row_hbm.at[pl.ds(base, C)], rowi)
        pltpu.sync_copy(p_hbm.at[pl.ds((w * NCH + ci) * CH, CH)], pbuf)
        for g in range(C // L):
            rbase = rowi[pl.ds(g * L, L)] * H
            for h in range(H):
                sv = plsc.load_gather(invb, [rbase + h])
                pnbuf[pl.ds(h * C + g * L, L)] = pbuf[pl.ds(h * C + g * L, L)] * sv
        pltpu.sync_copy(pnbuf, pn_hbm.at[pl.ds((w * NCH + ci) * CH, CH)])
        return carry

    lax.fori_loop(0, NCH, chunk, 0)


@functools.partial(
    pl.kernel,
    out_type=jax.ShapeDtypeStruct((NC, N, DC), jnp.float32),
    mesh=_mesh,
    compiler_params=_sc_params,
    scratch_types=[
        pltpu.VMEM((C,), jnp.int32),
        pltpu.VMEM((C,), jnp.int32),
        pltpu.VMEM((C, D), jnp.float32),
        pltpu.VMEM((C, DC), jnp.float32),
        pltpu.VMEM((HH * C,), jnp.float32),
        pltpu.VMEM((_ZR, DC), jnp.float32),
        pltpu.VMEM_SHARED((N, DC), jnp.float32),
        pltpu.SemaphoreType.DMA,
    ],
)
def _k5(v_hbm, row_hbm, col_hbm, pn_hbm,
        outp_hbm,
        rowi, coli, vrows, whbuf, pnbuf, zerobuf, shared_out, semv):
    c = lax.axis_index("c")
    s = lax.axis_index("s")
    iota = lax.broadcasted_iota(jnp.int32, (L,), 0)
    zero16 = jnp.zeros((L,), jnp.float32)

    def zrow(i, carry):
        for kk in range(DC // L):
            zerobuf[i, pl.ds(kk * L, L)] = zero16
        return carry

    lax.fori_loop(0, _ZR, zrow, 0)

    nblk = jnp.where(s == NS - 1, 5, 8)

    def zcopy(i, carry):
        pltpu.sync_copy(zerobuf, shared_out.at[pl.ds(s * 640 + i * _ZR, _ZR), :])
        return carry

    lax.fori_loop(0, nblk, zcopy, 0)
    plsc.subcore_barrier()

    def chunk(ci, carry):
        base = s * EPT + ci * C
        pltpu.sync_copy(row_hbm.at[pl.ds(base, C)], rowi)
        pltpu.sync_copy(col_hbm.at[pl.ds(base, C)], coli)
        cpv = pltpu.async_copy(v_hbm.at[coli], vrows, semv)
        j = s * NCH5 + ci
        pltpu.sync_copy(pn_hbm.at[pl.ds(j * CH + c * (HH * C), HH * C)], pnbuf)
        cpv.wait()
        for g in range(C // L):
            el = iota + (g * L)
            for h in range(HH):
                cv = pnbuf[pl.ds(h * C + g * L, L)]
                dsrc0 = (c * HH + h) * Dh

                def dstep(d, carry2):
                    dvs = jnp.full((L,), 0, jnp.int32) + (dsrc0 + d)
                    dvd = jnp.full((L,), h * Dh, jnp.int32) + d
                    colv = plsc.load_gather(vrows, [el, dvs])
                    plsc.store_scatter(whbuf, [el, dvd], colv * cv)
                    return carry2

                lax.fori_loop(0, Dh, dstep, 0, unroll=4)
        pltpu.sync_copy(whbuf, shared_out.at[rowi], add=True)
        return carry

    lax.fori_loop(0, NCH5, chunk, 0)
    plsc.subcore_barrier()

    @pl.when(s == 0)
    def _():
        pltpu.sync_copy(shared_out, outp_hbm.at[c])


def kernel(x, edge_index, edge_vec, edge_length, Wq, bq, Wk, bk, Wv, bv,
           W1, b1, W2, b2, Wo, bo, gamma, beta):
    row = edge_index[0]
    col = edge_index[1]
    q, k, v = _qkv(x, Wq, bq, Wk, bk, Wv, bv)
    c0, c1 = _edge_coeffs(edge_length, W1, b1, W2, b2)
    a, wmax = _k1(q, k, row, col, c0.reshape(E * H), c1.reshape(E))
    p, sums = _k3(row, a, wmax)
    inv = _k4a(sums)
    pn = _k4b(row, p, inv)
    outp = _k5(v, row, col, pn)
    acc = jnp.concatenate([outp[0], outp[1]], axis=1)
    return _final(acc, x, Wo, bo, gamma, beta)


# speedup vs baseline: 8.8154x; 1.0604x over previous
import functools
import math

import jax
import jax.numpy as jnp
from jax import lax
from jax.experimental import pallas as pl
from jax.experimental.pallas import tpu as pltpu
from jax.experimental.pallas import tpu_sc as plsc

N = 10000
E = 320000
D = 128
H = 8
Dh = D // H
CUTOFF = 5.0

NC = 2
NS = 16
NW = NC * NS
L = 16

EPW = E // NW
C = 80
NCH = EPW // C
NJ = E // C
SUMW = N * H
CH = C * H

_RED_W = 20
_RED_SL = SUMW // _RED_W

HH = H // NC
DC = D // NC
EPT = E // NS
NCH5 = EPT // C
_ZR = 80

_mesh = plsc.VectorSubcoreMesh(
    core_axis_name="c", subcore_axis_name="s", num_cores=NC, num_subcores=NS)
_sc_params = pltpu.CompilerParams(
    needs_layout_passes=False, use_tc_tiling_on_sc=False)


def _wid():
    return lax.axis_index("s") * NC + lax.axis_index("c")


_NBLK = 400


def _qkv_body(x_ref, wq_ref, bq_ref, wk_ref, bk_ref, wv_ref, bv_ref,
              q_ref, k_ref, v_ref):
    x = x_ref[...]
    q_ref[...] = jnp.dot(x, wq_ref[...], preferred_element_type=jnp.float32) + bq_ref[...]
    k_ref[...] = jnp.dot(x, wk_ref[...], preferred_element_type=jnp.float32) + bk_ref[...]
    v_ref[...] = jnp.dot(x, wv_ref[...], preferred_element_type=jnp.float32) + bv_ref[...]


def _qkv(x, Wq, bq, Wk, bk, Wv, bv):
    blk = pl.BlockSpec((_NBLK, D), lambda i: (i, 0))
    wblk = pl.BlockSpec((D, D), lambda i: (0, 0))
    bblk = pl.BlockSpec((1, D), lambda i: (0, 0))
    out = jax.ShapeDtypeStruct((N, D), jnp.float32)
    return pl.pallas_call(
        _qkv_body,
        grid=(N // _NBLK,),
        in_specs=[blk, wblk, bblk, wblk, bblk, wblk, bblk],
        out_specs=[blk, blk, blk],
        out_shape=[out, out, out],
    )(x, Wq, bq.reshape(1, D), Wk, bk.reshape(1, D), Wv, bv.reshape(1, D))


_EB = 2000


def _edge_body(len_ref, w1_ref, b1_ref, w2_ref, b2_ref, c0_ref, c1_ref):
    ln = len_ref[...]
    hid = jax.nn.silu(ln * w1_ref[...] + b1_ref[...])
    bias = jnp.dot(hid, w2_ref[...], preferred_element_type=jnp.float32) + b2_ref[...]
    cut = 0.5 * (jnp.cos(ln * (math.pi / CUTOFF)) + 1.0)
    cut = cut * (ln < CUTOFF).astype(jnp.float32)
    c0_ref[...] = bias * cut
    c1_ref[...] = cut * (1.0 / math.sqrt(Dh))


def _edge_coeffs(edge_length, W1, b1, W2, b2):
    lblk = pl.BlockSpec((_EB, 1), lambda i: (i, 0))
    w1blk = pl.BlockSpec((1, D), lambda i: (0, 0))
    w2blk = pl.BlockSpec((D, H), lambda i: (0, 0))
    b2blk = pl.BlockSpec((1, H), lambda i: (0, 0))
    return pl.pallas_call(
        _edge_body,
        grid=(E // _EB,),
        in_specs=[lblk, w1blk, w1blk, w2blk, b2blk],
        out_specs=[pl.BlockSpec((_EB, H), lambda i: (i, 0)), lblk],
        out_shape=[jax.ShapeDtypeStruct((E, H), jnp.float32),
                   jax.ShapeDtypeStruct((E, 1), jnp.float32)],
    )(edge_length, W1, b1.reshape(1, D), W2, b2.reshape(1, H))


def _final_body(acc_ref, x_ref, wo_ref, bo_ref, g_ref, b_ref, y_ref):
    o = jnp.dot(acc_ref[...], wo_ref[...], preferred_element_type=jnp.float32)
    y = o + bo_ref[...] + x_ref[...]
    mu = jnp.mean(y, axis=-1, keepdims=True)
    yc = y - mu
    var = jnp.mean(yc * yc, axis=-1, keepdims=True)
    yn = yc * lax.rsqrt(var + 1e-05)
    y_ref[...] = yn * g_ref[...] + b_ref[...]


def _final(acc, x, Wo, bo, gamma, beta):
    blk = pl.BlockSpec((_NBLK, D), lambda i: (i, 0))
    wblk = pl.BlockSpec((D, D), lambda i: (0, 0))
    bblk = pl.BlockSpec((1, D), lambda i: (0, 0))
    return pl.pallas_call(
        _final_body,
        grid=(N // _NBLK,),
        in_specs=[blk, blk, wblk, bblk, bblk, bblk],
        out_specs=blk,
        out_shape=jax.ShapeDtypeStruct((N, D), jnp.float32),
    )(acc, x, Wo, bo.reshape(1, D), gamma.reshape(1, D), beta.reshape(1, D))


@functools.partial(
    pl.kernel,
    out_type=[jax.ShapeDtypeStruct((E * H,), jnp.float32),
              jax.ShapeDtypeStruct((NW * H * L,), jnp.float32)],
    mesh=_mesh,
    compiler_params=_sc_params,
    scratch_types=[
        pltpu.VMEM((EPW,), jnp.int32),
        pltpu.VMEM((EPW,), jnp.int32),
        pltpu.VMEM((2, C, D), jnp.float32),
        pltpu.VMEM((2, C, D), jnp.float32),
        pltpu.VMEM((2, CH,), jnp.float32),
        pltpu.VMEM((2, CH,), jnp.float32),
        pltpu.VMEM((2, C), jnp.float32),
        pltpu.VMEM((H * L,), jnp.float32),
        pltpu.SemaphoreType.DMA,
        pltpu.SemaphoreType.DMA,
        pltpu.SemaphoreType.DMA,
        pltpu.SemaphoreType.DMA,
        pltpu.SemaphoreType.DMA,
        pltpu.SemaphoreType.DMA,
        pltpu.SemaphoreType.DMA,
        pltpu.SemaphoreType.DMA,
    ],
)
def _k1(q_hbm, k_hbm, row_hbm, col_hbm, c0_hbm, c1_hbm,
        a_hbm, wmax_hbm,
        rowi, coli, qrows, krows, abuf, c0buf, c1buf, wmaxb,
        semq0, semq1, semk0, semk1, semc0, semc1, semo0, semo1):
    w = _wid()
    iota = lax.broadcasted_iota(jnp.int32, (L,), 0)
    neg = jnp.full((L,), -3.0e38, jnp.float32)
    semq = [semq0, semq1]
    semk = [semk0, semk1]
    semc = [semc0, semc1]
    semo = [semo0, semo1]
    for h in range(H):
        wmaxb[pl.ds(h * L, L)] = neg

    base_w = w * EPW
    pltpu.sync_copy(row_hbm.at[pl.ds(base_w, EPW)], rowi)
    pltpu.sync_copy(col_hbm.at[pl.ds(base_w, EPW)], coli)

    def fire(ci, k):
        cc = jnp.minimum(ci, NCH - 1)
        pltpu.async_copy(q_hbm.at[rowi.at[pl.ds(cc * C, C)]],
                         qrows.at[k], semq[k])
        pltpu.async_copy(k_hbm.at[coli.at[pl.ds(cc * C, C)]],
                         krows.at[k], semk[k])
        pltpu.async_copy(c0_hbm.at[pl.ds((base_w + cc * C) * H, CH)],
                         c0buf.at[k], semc[k])
        pltpu.async_copy(c1_hbm.at[pl.ds(base_w + cc * C, C)],
                         c1buf.at[k], semc[k])

    def drain(k):
        pltpu.make_async_copy(q_hbm.at[rowi.at[pl.ds(0, C)]],
                              qrows.at[k], semq[k]).wait()
        pltpu.make_async_copy(k_hbm.at[coli.at[pl.ds(0, C)]],
                              krows.at[k], semk[k]).wait()
        pltpu.make_async_copy(c0_hbm.at[pl.ds(0, CH)], c0buf.at[k],
                              semc[k]).wait()
        pltpu.make_async_copy(c1_hbm.at[pl.ds(0, C)], c1buf.at[k],
                              semc[k]).wait()

    def wback(ci, k):
        cc = jnp.minimum(ci, NCH - 1)
        pltpu.async_copy(abuf.at[k], a_hbm.at[pl.ds((w * NCH + cc) * CH, CH)],
                         semo[k])

    def wback_wait(k):
        pltpu.make_async_copy(abuf.at[k], a_hbm.at[pl.ds(0, CH)],
                              semo[k]).wait()

    def compute(ci, k):
        qref = qrows.at[k]
        kref = krows.at[k]
        for g in range(C // L):
            el = iota + (g * L)
            el8 = el * H
            c1v = c1buf[k, pl.ds(g * L, L)]
            for h in range(H):
                def dstep(d, acc):
                    dv = jnp.full((L,), h * Dh, jnp.int32) + d
                    qv = plsc.load_gather(qref, [el, dv])
                    kv = plsc.load_gather(kref, [el, dv])
                    return acc + qv * kv
                dot = lax.fori_loop(0, Dh, dstep, jnp.zeros((L,), jnp.float32),
                                    unroll=4)
                c0v = plsc.load_gather(c0buf.at[k], [el8 + h])
                a = dot * c1v + c0v
                abuf[k, pl.ds(h * C + g * L, L)] = a
                wmaxb[pl.ds(h * L, L)] = jnp.maximum(wmaxb[pl.ds(h * L, L)], a)

    fire(0, 0)
    fire(1, 1)
    wback(0, 0)
    wback(1, 1)

    def body(i, carry):
        ci = 2 * i
        for k in range(2):
            drain(k)
            wback_wait(k)
            compute(ci + k, k)
            fire(ci + k + 2, k)
            wback(ci + k, k)
        return carry

    lax.fori_loop(0, NCH // 2, body, 0)
    ci = NCH - 1
    drain(0)
    wback_wait(0)
    compute(ci, 0)
    wback(ci, 0)
    drain(1)
    wback_wait(1)
    wback_wait(0)
    pltpu.sync_copy(wmaxb, wmax_hbm.at[pl.ds(w * H * L, H * L)])


@functools.partial(
    pl.kernel,
    out_type=[jax.ShapeDtypeStruct((E * H,), jnp.float32),
              jax.ShapeDtypeStruct((NW * SUMW,), jnp.float32)],
    mesh=_mesh,
    compiler_params=_sc_params,
    scratch_types=[
        pltpu.VMEM((C,), jnp.int32),
        pltpu.VMEM((CH,), jnp.float32),
        pltpu.VMEM((CH,), jnp.float32),
        pltpu.VMEM((NW * H * L,), jnp.float32),
        pltpu.VMEM((SUMW,), jnp.float32),
    ],
)
def _k3(row_hbm, a_hbm, wmax_hbm,
        p_hbm, sums_hbm,
        rowi, abuf, pbuf, wmaxall, sums):
    w = _wid()
    pltpu.sync_copy(wmax_hbm, wmaxall)
    m = []
    for h in range(H):
        acc = wmaxall[pl.ds(h * L, L)]
        for t in range(1, NW):
            acc = jnp.maximum(acc, wmaxall[pl.ds(t * H * L + h * L, L)])
        m.append(jnp.max(acc))

    zero16 = jnp.zeros((L,), jnp.float32)

    def zstep(i, carry):
        sums[pl.ds(i * L, L)] = zero16
        return carry

    lax.fori_loop(0, SUMW // L, zstep, 0)

    def chunk(ci, carry):
        base = w * EPW + ci * C
        pltpu.sync_copy(row_hbm.at[pl.ds(base, C)], rowi)
        pltpu.sync_copy(a_hbm.at[pl.ds((w * NCH + ci) * CH, CH)], abuf)
        for g in range(C // L):
            rbase = rowi[pl.ds(g * L, L)] * H
            for h in range(H):
                p = jnp.exp(abuf[pl.ds(h * C + g * L, L)] - m[h])
                pbuf[pl.ds(h * C + g * L, L)] = p
                plsc.addupdate_scatter(sums, [rbase + h], p)
        pltpu.sync_copy(pbuf, p_hbm.at[pl.ds((w * NCH + ci) * CH, CH)])
        return carry

    lax.fori_loop(0, NCH, chunk, 0)
    pltpu.sync_copy(sums, sums_hbm.at[pl.ds(w * SUMW, SUMW)])


@functools.partial(
    pl.kernel,
    out_type=jax.ShapeDtypeStruct((SUMW,), jnp.float32),
    mesh=_mesh,
    compiler_params=_sc_params,
    scratch_types=[
        pltpu.VMEM((_RED_SL,), jnp.float32),
        pltpu.VMEM((_RED_SL,), jnp.float32),
    ],
)
def _k4a(sums_hbm, inv_hbm, acc, tbuf):
    w = _wid()

    @pl.when(w < _RED_W)
    def _():
        base = w * _RED_SL
        ng = _RED_SL // L

        def zstep(i, carry):
            acc[pl.ds(i * L, L)] = jnp.zeros((L,), jnp.float32)
            return carry

        lax.fori_loop(0, ng, zstep, 0)

        def tstep(t, carry):
            pltpu.sync_copy(sums_hbm.at[pl.ds(t * SUMW + base, _RED_SL)], tbuf)

            def astep(i, c2):
                acc[pl.ds(i * L, L)] = acc[pl.ds(i * L, L)] + tbuf[pl.ds(i * L, L)]
                return c2

            lax.fori_loop(0, ng, astep, 0)
            return carry

        lax.fori_loop(0, NW, tstep, 0)

        def istep(i, carry):
            acc[pl.ds(i * L, L)] = 1.0 / (acc[pl.ds(i * L, L)] + 1e-08)
            return carry

        lax.fori_loop(0, ng, istep, 0)
        pltpu.sync_copy(acc, inv_hbm.at[pl.ds(base, _RED_SL)])


@functools.partial(
    pl.kernel,
    out_type=jax.ShapeDtypeStruct((E * H,), jnp.float32),
    mesh=_mesh,
    compiler_params=_sc_params,
    scratch_types=[
        pltpu.VMEM((C,), jnp.int32),
        pltpu.VMEM((CH,), jnp.float32),
        pltpu.VMEM((CH,), jnp.float32),
        pltpu.VMEM((SUMW,), jnp.float32),
    ],
)
def _k4b(row_hbm, p_hbm, inv_hbm, pn_hbm, rowi, pbuf, pnbuf, invb):
    w = _wid()
    pltpu.sync_copy(inv_hbm, invb)

    def chunk(ci, carry):
        base = w * EPW + ci * C
        pltpu.sync_copy(row_hbm.at[pl.ds(base, C)], rowi)
        pltpu.sync_copy(p_hbm.at[pl.ds((w * NCH + ci) * CH, CH)], pbuf)
        for g in range(C // L):
            rbase = rowi[pl.ds(g * L, L)] * H
            for h in range(H):
                sv = plsc.load_gather(invb, [rbase + h])
                pnbuf[pl.ds(h * C + g * L, L)] = pbuf[pl.ds(h * C + g * L, L)] * sv
        pltpu.sync_copy(pnbuf, pn_hbm.at[pl.ds((w * NCH + ci) * CH, CH)])
        return carry

    lax.fori_loop(0, NCH, chunk, 0)


@functools.partial(
    pl.kernel,
    out_type=jax.ShapeDtypeStruct((NC, N, DC), jnp.float32),
    mesh=_mesh,
    compiler_params=_sc_params,
    scratch_types=[
        pltpu.VMEM((C,), jnp.int32),
        pltpu.VMEM((C,), jnp.int32),
        pltpu.VMEM((C, D), jnp.float32),
        pltpu.VMEM((C, DC), jnp.float32),
        pltpu.VMEM((HH * C,), jnp.float32),
        pltpu.VMEM((_ZR, DC), jnp.float32),
        pltpu.VMEM_SHARED((N, DC), jnp.float32),
        pltpu.SemaphoreType.DMA,
    ],
)
def _k5(v_hbm, row_hbm, col_hbm, pn_hbm,
        outp_hbm,
        rowi, coli, vrows, whbuf, pnbuf, zerobuf, shared_out, semv):
    c = lax.axis_index("c")
    s = lax.axis_index("s")
    iota = lax.broadcasted_iota(jnp.int32, (L,), 0)
    zero16 = jnp.zeros((L,), jnp.float32)

    def zrow(i, carry):
        for kk in range(DC // L):
            zerobuf[i, pl.ds(kk * L, L)] = zero16
        return carry

    lax.fori_loop(0, _ZR, zrow, 0)

    nblk = jnp.where(s == NS - 1, 5, 8)

    def zcopy(i, carry):
        pltpu.sync_copy(zerobuf, shared_out.at[pl.ds(s * 640 + i * _ZR, _ZR), :])
        return carry

    lax.fori_loop(0, nblk, zcopy, 0)
    plsc.subcore_barrier()

    def chunk(ci, carry):
        base = s * EPT + ci * C
        pltpu.sync_copy(row_hbm.at[pl.ds(base, C)], rowi)
        pltpu.sync_copy(col_hbm.at[pl.ds(base, C)], coli)
        cpv = pltpu.async_copy(v_hbm.at[coli], vrows, semv)
        j = s * NCH5 + ci
        pltpu.sync_copy(pn_hbm.at[pl.ds(j * CH + c * (HH * C), HH * C)], pnbuf)
        cpv.wait()
        for g in range(C // L):
            el = iota + (g * L)
            for h in range(HH):
                cv = pnbuf[pl.ds(h * C + g * L, L)]
                dsrc0 = (c * HH + h) * Dh

                def dstep(d, carry2):
                    dvs = jnp.full((L,), 0, jnp.int32) + (dsrc0 + d)
                    dvd = jnp.full((L,), h * Dh, jnp.int32) + d
                    colv = plsc.load_gather(vrows, [el, dvs])
                    plsc.store_scatter(whbuf, [el, dvd], colv * cv)
                    return carry2

                lax.fori_loop(0, Dh, dstep, 0, unroll=4)
        pltpu.sync_copy(whbuf, shared_out.at[rowi], add=True)
        return carry

    lax.fori_loop(0, NCH5, chunk, 0)
    plsc.subcore_barrier()

    @pl.when(s == 0)
    def _():
        pltpu.sync_copy(shared_out, outp_hbm.at[c])


def kernel(x, edge_index, edge_vec, edge_length, Wq, bq, Wk, bk, Wv, bv,
           W1, b1, W2, b2, Wo, bo, gamma, beta):
    row = edge_index[0]
    col = edge_index[1]
    q, k, v = _qkv(x, Wq, bq, Wk, bk, Wv, bv)
    c0, c1 = _edge_coeffs(edge_length, W1, b1, W2, b2)
    a, wmax = _k1(q, k, row, col, c0.reshape(E * H), c1.reshape(E))
    p, sums = _k3(row, a, wmax)
    inv = _k4a(sums)
    pn = _k4b(row, p, inv)
    outp = _k5(v, row, col, pn)
    acc = jnp.concatenate([outp[0], outp[1]], axis=1)
    return _final(acc, x, Wo, bo, gamma, beta)
